# Initial kernel scaffold; baseline (speedup 1.0000x reference)
import jax
import jax.numpy as jnp
from jax.experimental import pallas as pl


def kernel(sequences, sequence_legths, table):
    raise NotImplementedError("write your pallas kernel here")


# speedup vs baseline: 1.3064x; 1.3064x over previous
import functools

import jax
import jax.numpy as jnp
from jax import lax
from jax.experimental import pallas as pl
from jax.experimental.pallas import tpu as pltpu
from jax.experimental.pallas import tpu_sc as plsc

NC = 2
NS = 16
NW = NC * NS

B = 4096
H = 200
D = 32
TOTAL = B * H
PER_TILE = TOTAL // NW
CHUNK = 128
NCHUNK = PER_TILE // CHUNK

_mesh = plsc.VectorSubcoreMesh(
    core_axis_name="c", subcore_axis_name="s", num_cores=NC, num_subcores=NS)


@functools.partial(
    pl.kernel,
    out_type=jax.ShapeDtypeStruct((NW, NCHUNK, CHUNK, D), jnp.float32),
    mesh=_mesh,
    scratch_types=[
        pltpu.VMEM((NCHUNK, CHUNK), jnp.int32),
        pltpu.VMEM((CHUNK, D), jnp.float32),
        pltpu.SemaphoreType.DMA,
    ],
    compiler_params=pltpu.CompilerParams(use_tc_tiling_on_sc=False),
)
def _gather_kernel(idx_hbm, table_hbm, out_hbm, idx_v, rows_v, gsem):
    wid = lax.axis_index("s") * NC + lax.axis_index("c")
    pltpu.sync_copy(idx_hbm.at[wid], idx_v)

    def chunk_body(j, carry):
        pltpu.async_copy(table_hbm.at[idx_v.at[j]], rows_v, gsem).wait()
        pltpu.sync_copy(rows_v, out_hbm.at[wid, j])
        return carry

    lax.fori_loop(0, NCHUNK, chunk_body, 0)


def kernel(sequences, sequence_legths, table):
    idx = sequences.reshape(NW, NCHUNK, CHUNK).astype(jnp.int32)
    out = _gather_kernel(idx, table)
    return out.reshape(B, H, D)


# speedup vs baseline: 1.4959x; 1.1451x over previous
import functools

import jax
import jax.numpy as jnp
from jax import lax
from jax.experimental import pallas as pl
from jax.experimental.pallas import tpu as pltpu
from jax.experimental.pallas import tpu_sc as plsc

NC = 2
NS = 16
NW = NC * NS

B = 4096
H = 200
D = 32
TOTAL = B * H
PER_TILE = TOTAL // NW
CHUNK = 128
NCHUNK = PER_TILE // CHUNK
K = 10
G = NCHUNK // K

_mesh = plsc.VectorSubcoreMesh(
    core_axis_name="c", subcore_axis_name="s", num_cores=NC, num_subcores=NS)


@functools.partial(
    pl.kernel,
    out_type=jax.ShapeDtypeStruct((NW, NCHUNK, CHUNK, D), jnp.float32),
    mesh=_mesh,
    scratch_types=[
        pltpu.VMEM((NCHUNK, CHUNK), jnp.int32),
        pltpu.VMEM((2, K, CHUNK, D), jnp.float32),
        pltpu.SemaphoreType.DMA,
        pltpu.SemaphoreType.DMA,
        pltpu.SemaphoreType.DMA,
    ],
    compiler_params=pltpu.CompilerParams(use_tc_tiling_on_sc=False),
)
def _gather_kernel(idx_hbm, table_hbm, out_hbm, idx_v, rows_v, gsem0, gsem1,
                   osem):
    wid = lax.axis_index("s") * NC + lax.axis_index("c")
    pltpu.sync_copy(idx_hbm.at[wid], idx_v)

    gsems = (gsem0, gsem1)

    def fire_gathers(g, p):
        for b in range(K):
            pltpu.async_copy(
                table_hbm.at[idx_v.at[g * K + b]], rows_v.at[p, b], gsems[p])

    def drain_gathers(p):
        pltpu.make_async_copy(
            out_hbm.at[0, pl.ds(0, K)], rows_v.at[p], gsems[p]).wait()

    def wait_one_out():
        pltpu.make_async_copy(
            rows_v.at[0], out_hbm.at[0, pl.ds(0, K)], osem).wait()

    def step(g, p):
        drain_gathers(p)
        @pl.when(g >= 1)
        def _():
            wait_one_out()

        pltpu.async_copy(rows_v.at[p], out_hbm.at[wid, pl.ds(g * K, K)], osem)

        @pl.when(g + 1 < G)
        def _():
            fire_gathers(g + 1, 1 - p)

    fire_gathers(0, 0)

    def pair(i, carry):
        step(2 * i, 0)
        step(2 * i + 1, 1)
        return carry

    lax.fori_loop(0, G // 2, pair, 0)
    wait_one_out()


def kernel(sequences, sequence_legths, table):
    idx = sequences.reshape(NW, NCHUNK, CHUNK).astype(jnp.int32)
    out = _gather_kernel(idx, table)
    return out.reshape(B, H, D)
